# baseline (speedup 1.0000x reference)
import jax
import jax.numpy as jnp
from jax.experimental import pallas as pl
from jax.experimental.pallas import tpu as pltpu

N_CHANNELS = 4096
N_ENERGIES = 16384
BC = 128
HALF_BLOCKS = N_CHANNELS // 2 // BC


def _body(params_ref, en_ref, tma_ref, tmb_ref, outa_ref, outb_ref, flux_ref):
    @pl.when(pl.program_id(0) == 0)
    def _():
        alpha = params_ref[0] + 1.2
        norm = params_ref[1]
        one_m_a = 1.0 - alpha
        e_low = en_ref[0:1, :]
        e_high = en_ref[1:2, :]
        flux_ref[...] = norm * (
            jnp.power(e_high, one_m_a) - jnp.power(e_low, one_m_a)
        ) / one_m_a

    flux = flux_ref[...]
    acca = jnp.sum(tma_ref[...] * flux, axis=1)
    accb = jnp.sum(tmb_ref[...] * flux, axis=1)
    outa_ref[...] = jnp.maximum(acca, 1e-6).reshape(1, 1, BC)
    outb_ref[...] = jnp.maximum(accb, 1e-6).reshape(1, 1, BC)


def kernel(parameters, energies, transfer_matrix):
    outa, outb = pl.pallas_call(
        _body,
        grid=(HALF_BLOCKS,),
        in_specs=[
            pl.BlockSpec(memory_space=pltpu.SMEM),
            pl.BlockSpec((2, N_ENERGIES), lambda i: (0, 0)),
            pl.BlockSpec((BC, N_ENERGIES), lambda i: (i, 0)),
            pl.BlockSpec((BC, N_ENERGIES), lambda i: (i + HALF_BLOCKS, 0)),
        ],
        out_specs=[
            pl.BlockSpec((1, 1, BC), lambda i: (i, 0, 0)),
            pl.BlockSpec((1, 1, BC), lambda i: (i, 0, 0)),
        ],
        out_shape=[
            jax.ShapeDtypeStruct((HALF_BLOCKS, 1, BC), jnp.float32),
            jax.ShapeDtypeStruct((HALF_BLOCKS, 1, BC), jnp.float32),
        ],
        scratch_shapes=[pltpu.VMEM((1, N_ENERGIES), jnp.float32)],
    )(parameters, energies, transfer_matrix, transfer_matrix)
    return jnp.concatenate(
        [outa.reshape(N_CHANNELS // 2), outb.reshape(N_CHANNELS // 2)])


# speedup vs baseline: 1.0320x; 1.0320x over previous
import jax
import jax.numpy as jnp
from jax.experimental import pallas as pl
from jax.experimental.pallas import tpu as pltpu

N_CHANNELS = 4096
N_ENERGIES = 16384
BC = 128


def _body(params_ref, en_ref, tm_ref, out_ref, flux_ref):
    i = pl.program_id(0)

    @pl.when(i == 0)
    def _():
        alpha = params_ref[0] + 1.2
        norm = params_ref[1]
        one_m_a = 1.0 - alpha
        e_low = en_ref[0:1, :]
        e_high = en_ref[1:2, :]
        flux_ref[...] = norm * (
            jnp.power(e_high, one_m_a) - jnp.power(e_low, one_m_a)
        ) / one_m_a

    flux = flux_ref[...]
    acc = jnp.sum(tm_ref[...] * flux, axis=1)
    out_ref[...] = jnp.maximum(acc, 1e-6).reshape(1, 1, BC)


def kernel(parameters, energies, transfer_matrix):
    out = pl.pallas_call(
        _body,
        grid=(N_CHANNELS // BC,),
        in_specs=[
            pl.BlockSpec(memory_space=pltpu.SMEM),
            pl.BlockSpec((2, N_ENERGIES), lambda i: (0, 0)),
            pl.BlockSpec((BC, N_ENERGIES), lambda i: (i, 0)),
        ],
        out_specs=pl.BlockSpec((1, 1, BC), lambda i: (i, 0, 0)),
        out_shape=jax.ShapeDtypeStruct((N_CHANNELS // BC, 1, BC), jnp.float32),
        scratch_shapes=[pltpu.VMEM((1, N_ENERGIES), jnp.float32)],
    )(parameters, energies, transfer_matrix)
    return out.reshape(N_CHANNELS)
